# baseline (speedup 1.0000x reference)
import functools

import jax
import jax.numpy as jnp
from jax import lax
from jax.experimental import pallas as pl
from jax.experimental.pallas import tpu as pltpu
from jax.experimental.pallas import tpu_sc as plsc

_NC = 2
_NS = 16
_NW = _NC * _NS
_L = 16


def _make_sc_kernel(batch, size, hp):
    rows = batch // _NW
    elems = rows * hp
    nchunk = elems // 128
    kpt = hp // _L
    cpr = hp // 128
    mesh = plsc.VectorSubcoreMesh(core_axis_name="c", subcore_axis_name="s")

    @functools.partial(
        pl.kernel,
        out_type=(
            jax.ShapeDtypeStruct((batch * hp,), jnp.float32),
            jax.ShapeDtypeStruct((batch * hp,), jnp.float32),
        ),
        mesh=mesh,
        scratch_types=[
            pltpu.VMEM((hp,), jnp.int32),
            pltpu.VMEM((nchunk, 128), jnp.int32),
            pltpu.VMEM((elems,), jnp.float32),
            pltpu.VMEM((elems,), jnp.float32),
            pltpu.SemaphoreType.DMA,
            pltpu.SemaphoreType.DMA,
        ],
    )
    def gather_sigmoid(w0, w1, idxp, o0, o1, idx_v, idxfull, v0, v1, sem0, sem1):
        wid = lax.axis_index("s") * _NC + lax.axis_index("c")
        row0 = wid * rows
        pltpu.sync_copy(idxp, idx_v)

        @pl.loop(0, rows)
        def _build(i):
            base = (row0 + i) * size
            for kk in range(kpt):
                chunk = idx_v[pl.ds(kk * _L, _L)] + base
                idxfull[i * cpr + kk // 8, pl.ds((kk % 8) * _L, _L)] = chunk

        @pl.loop(0, nchunk)
        def _fire(c):
            dst = pl.ds(c * 128, 128)
            pltpu.async_copy(w0.at[idxfull.at[c]], v0.at[dst], sem0)
            pltpu.async_copy(w1.at[idxfull.at[c]], v1.at[dst], sem1)

        @pl.loop(0, nchunk)
        def _drain(c):
            dst = pl.ds(c * 128, 128)
            pltpu.make_async_copy(w0.at[idxfull.at[c]], v0.at[dst], sem0).wait()
            pltpu.make_async_copy(w1.at[idxfull.at[c]], v1.at[dst], sem1).wait()

        @pl.loop(0, elems // (_L * 8))
        def _act(r):
            for kk in range(8):
                s = pl.ds((r * 8 + kk) * _L, _L)
                for v in (v0, v1):
                    x = v[s]
                    v[s] = 1.0 / (1.0 + jnp.exp(-2.0 * x))

        pltpu.sync_copy(v0, o0.at[pl.ds(wid * elems, elems)])
        pltpu.sync_copy(v1, o1.at[pl.ds(wid * elems, elems)])

    return gather_sigmoid


def kernel(W0, W1, idx):
    batch, size = W0.shape
    hist = idx.shape[0]
    hp = ((hist + 127) // 128) * 128
    idxp = jnp.zeros((hp,), jnp.int32).at[:hist].set(idx.astype(jnp.int32))
    sc = _make_sc_kernel(batch, size, hp)
    o0, o1 = sc(W0.reshape(-1), W1.reshape(-1), idxp)
    o0 = o0.reshape(batch, hp)[:, :hist, None]
    o1 = o1.reshape(batch, hp)[:, :hist, None]
    return (o0, o1)


# speedup vs baseline: 24.8736x; 24.8736x over previous
import functools

import jax
import jax.numpy as jnp
from jax import lax
from jax.experimental import pallas as pl
from jax.experimental.pallas import tpu as pltpu
from jax.experimental.pallas import tpu_sc as plsc

_NC = 2
_NS = 16
_NW = _NC * _NS
_L = 16


def _make_sc_kernel(batch, size, hp):
    cpt = hp // _NW
    kpb = batch // _L
    mesh = plsc.VectorSubcoreMesh(core_axis_name="c", subcore_axis_name="s")

    @functools.partial(
        pl.kernel,
        out_type=(
            jax.ShapeDtypeStruct((hp, batch), jnp.float32),
            jax.ShapeDtypeStruct((hp, batch), jnp.float32),
        ),
        mesh=mesh,
        scratch_types=[
            pltpu.VMEM((hp,), jnp.int32),
            pltpu.VMEM((cpt, batch), jnp.float32),
            pltpu.VMEM((cpt, batch), jnp.float32),
            pltpu.SemaphoreType.DMA,
            pltpu.SemaphoreType.DMA,
        ],
    )
    def gather_sigmoid(wt0, wt1, idxp, o0, o1, idx_v, col0, col1, sem0, sem1):
        wid = lax.axis_index("s") * _NC + lax.axis_index("c")
        j0 = wid * cpt
        pltpu.sync_copy(idxp, idx_v)

        my_idx = idx_v.at[pl.ds(j0, cpt)]
        pltpu.async_copy(wt0.at[my_idx], col0, sem0)
        pltpu.async_copy(wt1.at[my_idx], col1, sem1)
        pltpu.make_async_copy(wt0.at[my_idx], col0, sem0).wait()
        pltpu.make_async_copy(wt1.at[my_idx], col1, sem1).wait()

        @pl.loop(0, kpb)
        def _act(k):
            s = pl.ds(k * _L, _L)
            for c in range(cpt):
                for col in (col0, col1):
                    x = col[c, s]
                    col[c, s] = 1.0 / (1.0 + jnp.exp(-2.0 * x))

        pltpu.sync_copy(col0, o0.at[pl.ds(j0, cpt), :])
        pltpu.sync_copy(col1, o1.at[pl.ds(j0, cpt), :])

    return gather_sigmoid


def kernel(W0, W1, idx):
    batch, size = W0.shape
    hist = idx.shape[0]
    hp = ((hist + 8 * _NW - 1) // (8 * _NW)) * (8 * _NW)
    idxp = jnp.zeros((hp,), jnp.int32).at[:hist].set(idx.astype(jnp.int32))
    sc = _make_sc_kernel(batch, size, hp)
    o0, o1 = sc(W0.T, W1.T, idxp)
    o0 = o0[:hist].T[..., None]
    o1 = o1[:hist].T[..., None]
    return (o0, o1)


# speedup vs baseline: 25.6229x; 1.0301x over previous
import functools

import jax
import jax.numpy as jnp
from jax import lax
from jax.experimental import pallas as pl
from jax.experimental.pallas import tpu as pltpu
from jax.experimental.pallas import tpu_sc as plsc

_NC = 2
_NS = 16
_NW = _NC * _NS
_L = 16
_CPT = 8


def _make_sc_kernel(batch, size, hp):
    kpb = batch // _L
    mesh = plsc.VectorSubcoreMesh(core_axis_name="c", subcore_axis_name="s")

    @functools.partial(
        pl.kernel,
        out_type=(
            jax.ShapeDtypeStruct((hp, batch), jnp.float32),
            jax.ShapeDtypeStruct((hp, batch), jnp.float32),
        ),
        mesh=mesh,
        scratch_types=[
            pltpu.VMEM((_CPT,), jnp.int32),
            pltpu.VMEM((_CPT, batch), jnp.float32),
            pltpu.VMEM((_CPT, batch), jnp.float32),
            pltpu.VMEM((_CPT, batch), jnp.float32),
            pltpu.VMEM((_CPT, batch), jnp.float32),
            pltpu.SemaphoreType.DMA,
            pltpu.SemaphoreType.DMA,
        ],
    )
    def gather_sigmoid(wt0, wt1, idx, o0, o1, idx_v, g0, g1, ob0, ob1, sem0, sem1):
        wid = lax.axis_index("s") * _NC + lax.axis_index("c")
        j0 = wid * _CPT

        @pl.when(j0 < hp)
        def _active():
            pltpu.sync_copy(idx.at[pl.ds(j0, _CPT)], idx_v)
            pltpu.async_copy(wt0.at[idx_v], g0, sem0)
            pltpu.async_copy(wt1.at[idx_v], g1, sem1)

            def _act(g, ob):
                @pl.loop(0, kpb)
                def _(k):
                    s = pl.ds(k * _L, _L)
                    for c in range(_CPT):
                        ob[c, s] = 1.0 / (1.0 + jnp.exp(-2.0 * g[c, s]))

            pltpu.make_async_copy(wt0.at[idx_v], g0, sem0).wait()
            _act(g0, ob0)
            pltpu.make_async_copy(wt1.at[idx_v], g1, sem1).wait()
            pltpu.sync_copy(ob0, o0.at[pl.ds(j0, _CPT), :])
            _act(g1, ob1)
            pltpu.sync_copy(ob1, o1.at[pl.ds(j0, _CPT), :])

    return gather_sigmoid


def kernel(W0, W1, idx):
    batch, size = W0.shape
    hist = idx.shape[0]
    hp = ((hist + _CPT - 1) // _CPT) * _CPT
    idx32 = idx.astype(jnp.int32)
    if hp != hist:
        idx32 = jnp.concatenate([idx32, jnp.zeros((hp - hist,), jnp.int32)])
    sc = _make_sc_kernel(batch, size, hp)
    o0, o1 = sc(W0.T, W1.T, idx32)
    o0 = o0[:hist].T[..., None]
    o1 = o1[:hist].T[..., None]
    return (o0, o1)
